# baseline (speedup 1.0000x reference)
import functools

import jax
import jax.numpy as jnp
from jax import lax
from jax.experimental import pallas as pl
from jax.experimental.pallas import tpu as pltpu
from jax.experimental.pallas import tpu_sc as plsc

N_NODES = 10000
N_EDGES = 160000
D = 128
LANES = 16

NC = 2
NS = 16
NW = NC * NS
CHUNK = 128
N_CHUNKS = N_EDGES // CHUNK
MAX_T = (N_CHUNKS + NW - 1) // NW
N_PAD = 10112
ROWS_PER_SUB = N_PAD // NS

_MESH = plsc.VectorSubcoreMesh(core_axis_name="c", subcore_axis_name="s")


def _zero_accumulator(rows_v, acc_sh, s, stage_rows):
    def zrow(i, carry):
        for j in range(D // LANES):
            rows_v[i, pl.ds(j * LANES, LANES)] = jnp.zeros((LANES,), jnp.float32)
        return carry

    lax.fori_loop(0, stage_rows, zrow, 0)
    nfull, rem = divmod(ROWS_PER_SUB, stage_rows)
    for z in range(nfull):
        pltpu.sync_copy(
            rows_v,
            acc_sh.at[pl.ds(s * ROWS_PER_SUB + z * stage_rows, stage_rows)],
        )
    if rem:
        pltpu.sync_copy(
            rows_v.at[pl.ds(0, rem)],
            acc_sh.at[pl.ds(s * ROWS_PER_SUB + nfull * stage_rows, rem)],
        )


def _seg_sum_h_body(prev_h, seg, out, rows_v, idx_v, acc_sh, sem_ld, sem_ix, sem_sc):
    c = lax.axis_index("c")
    s = lax.axis_index("s")
    w = c * NS + s

    _zero_accumulator(rows_v.at[0], acc_sh, s, CHUNK)
    plsc.subcore_barrier()

    def start_loads(t, b):
        k = w + NW * t

        @pl.when(k < N_CHUNKS)
        def _():
            base = k * CHUNK
            pltpu.async_copy(prev_h.at[pl.ds(base, CHUNK)], rows_v.at[b], sem_ld.at[b])
            pltpu.async_copy(seg.at[pl.ds(base, CHUNK)], idx_v.at[b], sem_ix.at[b])

    start_loads(0, 0)

    def pair(t2, carry):
        for b in (0, 1):
            t = 2 * t2 + b
            k = w + NW * t

            @pl.when(k < N_CHUNKS)
            def _():
                pltpu.make_async_copy(
                    prev_h.at[pl.ds(0, CHUNK)], rows_v.at[b], sem_ld.at[b]
                ).wait()
                pltpu.make_async_copy(
                    seg.at[pl.ds(0, CHUNK)], idx_v.at[b], sem_ix.at[b]
                ).wait()

            @pl.when((t >= 1) & (k - NW < N_CHUNKS))
            def _():
                pltpu.make_async_copy(
                    rows_v.at[1 - b], acc_sh.at[idx_v.at[1 - b]], sem_sc.at[1 - b]
                ).wait()

            start_loads(t + 1, 1 - b)

            @pl.when(k < N_CHUNKS)
            def _():
                pltpu.async_copy(
                    rows_v.at[b], acc_sh.at[idx_v.at[b]], sem_sc.at[b], add=True
                )

        return carry

    lax.fori_loop(0, MAX_T // 2, pair, 0)
    @pl.when(w + NW * (MAX_T - 1) < N_CHUNKS)
    def _():
        pltpu.make_async_copy(
            rows_v.at[1], acc_sh.at[idx_v.at[1]], sem_sc.at[1]
        ).wait()

    plsc.subcore_barrier()
    pltpu.sync_copy(
        acc_sh.at[pl.ds(s * ROWS_PER_SUB, ROWS_PER_SUB)],
        out.at[c, pl.ds(s * ROWS_PER_SUB, ROWS_PER_SUB)],
    )


_seg_sum_h = functools.partial(
    pl.kernel,
    out_type=jax.ShapeDtypeStruct((NC, N_PAD, D), jnp.float32),
    mesh=_MESH,
    scratch_types=[
        pltpu.VMEM((2, CHUNK, D), jnp.float32),
        pltpu.VMEM((2, CHUNK), jnp.int32),
        pltpu.VMEM_SHARED((N_PAD, D), jnp.float32),
        pltpu.SemaphoreType.DMA((2,)),
        pltpu.SemaphoreType.DMA((2,)),
        pltpu.SemaphoreType.DMA((2,)),
    ],
)(_seg_sum_h_body)


def _fc_body(fh, pc, seg, fx, out, fh_v, pc_v, fxg_v, idx_v, acc_sh, sem, sem2, sem3):
    c = lax.axis_index("c")
    s = lax.axis_index("s")
    w = c * NS + s

    _zero_accumulator(fh_v, acc_sh, s, CHUNK)
    plsc.subcore_barrier()

    def body(t, carry):
        k = w + NW * t

        @pl.when(k < N_CHUNKS)
        def _():
            base = k * CHUNK
            cp1 = pltpu.async_copy(fh.at[pl.ds(base, CHUNK)], fh_v, sem)
            cp2 = pltpu.async_copy(pc.at[pl.ds(base, CHUNK)], pc_v, sem2)
            pltpu.sync_copy(seg.at[pl.ds(base, CHUNK)], idx_v)
            cp3 = pltpu.async_copy(fx.at[idx_v], fxg_v, sem3)
            cp1.wait()
            cp2.wait()
            cp3.wait()

            @plsc.parallel_loop(0, CHUNK, 1, unroll=8)
            def row(i):
                for j in range(D // LANES):
                    sl = pl.ds(j * LANES, LANES)
                    x = fh_v[i, sl] + fxg_v[i, sl]
                    sg = 1.0 / (1.0 + jnp.exp(-x))
                    fh_v[i, sl] = sg * pc_v[i, sl]

            pltpu.sync_copy(fh_v, acc_sh.at[idx_v], add=True)

        return carry

    lax.fori_loop(0, MAX_T, body, 0)
    plsc.subcore_barrier()
    pltpu.sync_copy(
        acc_sh.at[pl.ds(s * ROWS_PER_SUB, ROWS_PER_SUB)],
        out.at[c, pl.ds(s * ROWS_PER_SUB, ROWS_PER_SUB)],
    )


_fc_partials = functools.partial(
    pl.kernel,
    out_type=jax.ShapeDtypeStruct((NC, N_PAD, D), jnp.float32),
    mesh=_MESH,
    scratch_types=[
        pltpu.VMEM((CHUNK, D), jnp.float32),
        pltpu.VMEM((CHUNK, D), jnp.float32),
        pltpu.VMEM((CHUNK, D), jnp.float32),
        pltpu.VMEM((CHUNK,), jnp.int32),
        pltpu.VMEM_SHARED((N_PAD, D), jnp.float32),
        pltpu.SemaphoreType.DMA,
        pltpu.SemaphoreType.DMA,
        pltpu.SemaphoreType.DMA,
    ],
)(_fc_body)


def _fwd_matmuls_kernel(ph_ref, uf_ref, x_ref, wf_ref, bf_ref, fh_ref, fx_ref):
    i = pl.program_id(0)
    fh_ref[...] = jnp.dot(
        ph_ref[...], uf_ref[...], preferred_element_type=jnp.float32
    )

    @pl.when(i < 10)
    def _():
        fx_ref[...] = (
            jnp.dot(x_ref[...], wf_ref[...], preferred_element_type=jnp.float32)
            + bf_ref[...]
        )


def _final_kernel(x_ref, htp_ref, fcp_ref, wc_ref, bc_ref, c_ref, h_ref):
    ht = htp_ref[0] + htp_ref[1]
    big_in = jnp.concatenate([x_ref[...], ht], axis=1)
    big = (
        jnp.dot(big_in, wc_ref[...], preferred_element_type=jnp.float32)
        + bc_ref[...]
    )
    z_i = big[:, :D]
    z_o = big[:, D : 2 * D]
    z_u = big[:, 2 * D :]
    fc = fcp_ref[0] + fcp_ref[1]
    cc = jax.nn.sigmoid(z_i) * jnp.tanh(z_u) + fc
    c_ref[...] = cc
    h_ref[...] = jax.nn.sigmoid(z_o) * jnp.tanh(cc)


@jax.jit
def kernel(inputs_mat, prev_c_mat, prev_h_mat, segment_ids, W_combined,
           b_combined, W_f, U_f, b_f):
    seg = segment_ids.astype(jnp.int32)

    htp = _seg_sum_h(prev_h_mat, seg)

    f_h, f_x = pl.pallas_call(
        _fwd_matmuls_kernel,
        grid=(100,),
        in_specs=[
            pl.BlockSpec((1600, D), lambda i: (i, 0)),
            pl.BlockSpec((D, D), lambda i: (0, 0)),
            pl.BlockSpec((1000, D), lambda i: (jnp.minimum(i, 9), 0)),
            pl.BlockSpec((D, D), lambda i: (0, 0)),
            pl.BlockSpec((1, D), lambda i: (0, 0)),
        ],
        out_specs=[
            pl.BlockSpec((1600, D), lambda i: (i, 0)),
            pl.BlockSpec((1000, D), lambda i: (jnp.minimum(i, 9), 0)),
        ],
        out_shape=[
            jax.ShapeDtypeStruct((N_EDGES, D), jnp.float32),
            jax.ShapeDtypeStruct((N_NODES, D), jnp.float32),
        ],
    )(prev_h_mat, U_f, inputs_mat, W_f, b_f)

    fcp = _fc_partials(f_h, prev_c_mat, seg, f_x)

    c, h = pl.pallas_call(
        _final_kernel,
        grid=(10,),
        in_specs=[
            pl.BlockSpec((1000, D), lambda i: (i, 0)),
            pl.BlockSpec((NC, 1000, D), lambda i: (0, i, 0)),
            pl.BlockSpec((NC, 1000, D), lambda i: (0, i, 0)),
            pl.BlockSpec((2 * D, 3 * D), lambda i: (0, 0)),
            pl.BlockSpec((1, 3 * D), lambda i: (0, 0)),
        ],
        out_specs=[
            pl.BlockSpec((1000, D), lambda i: (i, 0)),
            pl.BlockSpec((1000, D), lambda i: (i, 0)),
        ],
        out_shape=[
            jax.ShapeDtypeStruct((N_NODES, D), jnp.float32),
            jax.ShapeDtypeStruct((N_NODES, D), jnp.float32),
        ],
    )(inputs_mat, htp, fcp, W_combined, b_combined)

    return (c, h)


# speedup vs baseline: 2.4175x; 2.4175x over previous
import functools

import jax
import jax.numpy as jnp
from jax import lax
from jax.experimental import pallas as pl
from jax.experimental.pallas import tpu as pltpu
from jax.experimental.pallas import tpu_sc as plsc

N_NODES = 10000
N_EDGES = 160000
D = 128
LANES = 16

NC = 2
NS = 16
NW = NC * NS
CHUNK = 128
N_CHUNKS = N_EDGES // CHUNK
MAX_T = (N_CHUNKS + NW - 1) // NW
N_PAD = 10112
ROWS_PER_SUB = N_PAD // NS

_MESH = plsc.VectorSubcoreMesh(core_axis_name="c", subcore_axis_name="s")


def _zero_accumulator(rows_v, acc_sh, s, stage_rows):
    def zrow(i, carry):
        for j in range(D // LANES):
            rows_v[i, pl.ds(j * LANES, LANES)] = jnp.zeros((LANES,), jnp.float32)
        return carry

    lax.fori_loop(0, stage_rows, zrow, 0)
    nfull, rem = divmod(ROWS_PER_SUB, stage_rows)
    for z in range(nfull):
        pltpu.sync_copy(
            rows_v,
            acc_sh.at[pl.ds(s * ROWS_PER_SUB + z * stage_rows, stage_rows)],
        )
    if rem:
        pltpu.sync_copy(
            rows_v.at[pl.ds(0, rem)],
            acc_sh.at[pl.ds(s * ROWS_PER_SUB + nfull * stage_rows, rem)],
        )


def _seg_sum_h_body(prev_h, seg, out, rows_v, idx_v, acc_sh, sem_ld, sem_ix, sem_sc):
    c = lax.axis_index("c")
    s = lax.axis_index("s")
    w = c * NS + s

    _zero_accumulator(rows_v.at[0], acc_sh, s, CHUNK)
    plsc.subcore_barrier()

    def start_loads(t, b):
        k = w + NW * t

        @pl.when(k < N_CHUNKS)
        def _():
            base = k * CHUNK
            pltpu.async_copy(prev_h.at[pl.ds(base, CHUNK)], rows_v.at[b], sem_ld.at[b])
            pltpu.async_copy(seg.at[pl.ds(base, CHUNK)], idx_v.at[b], sem_ix.at[b])

    start_loads(0, 0)

    def pair(t2, carry):
        for b in (0, 1):
            t = 2 * t2 + b
            k = w + NW * t

            @pl.when(k < N_CHUNKS)
            def _():
                pltpu.make_async_copy(
                    prev_h.at[pl.ds(0, CHUNK)], rows_v.at[b], sem_ld.at[b]
                ).wait()
                pltpu.make_async_copy(
                    seg.at[pl.ds(0, CHUNK)], idx_v.at[b], sem_ix.at[b]
                ).wait()

            @pl.when((t >= 1) & (k - NW < N_CHUNKS))
            def _():
                pltpu.make_async_copy(
                    rows_v.at[1 - b], acc_sh.at[idx_v.at[1 - b]], sem_sc.at[1 - b]
                ).wait()

            start_loads(t + 1, 1 - b)

            @pl.when(k < N_CHUNKS)
            def _():
                pltpu.async_copy(
                    rows_v.at[b], acc_sh.at[idx_v.at[b]], sem_sc.at[b], add=True
                )

        return carry

    lax.fori_loop(0, MAX_T // 2, pair, 0)
    @pl.when(w + NW * (MAX_T - 1) < N_CHUNKS)
    def _():
        pltpu.make_async_copy(
            rows_v.at[1], acc_sh.at[idx_v.at[1]], sem_sc.at[1]
        ).wait()

    plsc.subcore_barrier()
    pltpu.sync_copy(
        acc_sh.at[pl.ds(s * ROWS_PER_SUB, ROWS_PER_SUB)],
        out.at[c, pl.ds(s * ROWS_PER_SUB, ROWS_PER_SUB)],
    )


_seg_sum_h = functools.partial(
    pl.kernel,
    out_type=jax.ShapeDtypeStruct((NC, N_PAD, D), jnp.float32),
    mesh=_MESH,
    scratch_types=[
        pltpu.VMEM((2, CHUNK, D), jnp.float32),
        pltpu.VMEM((2, CHUNK), jnp.int32),
        pltpu.VMEM_SHARED((N_PAD, D), jnp.float32),
        pltpu.SemaphoreType.DMA((2,)),
        pltpu.SemaphoreType.DMA((2,)),
        pltpu.SemaphoreType.DMA((2,)),
    ],
)(_seg_sum_h_body)


def _fc_body(fh, pc, seg, fx, out, fh_v, pc_v, fxg_v, idx_v, acc_sh, sem, sem2, sem3):
    c = lax.axis_index("c")
    s = lax.axis_index("s")
    w = c * NS + s

    _zero_accumulator(fh_v, acc_sh, s, CHUNK)
    plsc.subcore_barrier()

    def body(t, carry):
        k = w + NW * t

        @pl.when(k < N_CHUNKS)
        def _():
            base = k * CHUNK
            cp1 = pltpu.async_copy(fh.at[pl.ds(base, CHUNK)], fh_v, sem)
            cp2 = pltpu.async_copy(pc.at[pl.ds(base, CHUNK)], pc_v, sem2)
            pltpu.sync_copy(seg.at[pl.ds(base, CHUNK)], idx_v)
            cp3 = pltpu.async_copy(fx.at[idx_v], fxg_v, sem3)
            cp1.wait()
            cp2.wait()
            cp3.wait()

            @plsc.parallel_loop(0, CHUNK, 1, unroll=4)
            def row(i):
                for j in range(D // LANES):
                    sl = pl.ds(j * LANES, LANES)
                    x = fh_v[i, sl] + fxg_v[i, sl]
                    sg = 1.0 / (1.0 + jnp.exp(-x))
                    fh_v[i, sl] = sg * pc_v[i, sl]

            pltpu.sync_copy(fh_v, acc_sh.at[idx_v], add=True)

        return carry

    lax.fori_loop(0, MAX_T, body, 0)
    plsc.subcore_barrier()
    pltpu.sync_copy(
        acc_sh.at[pl.ds(s * ROWS_PER_SUB, ROWS_PER_SUB)],
        out.at[c, pl.ds(s * ROWS_PER_SUB, ROWS_PER_SUB)],
    )


_fc_partials = functools.partial(
    pl.kernel,
    out_type=jax.ShapeDtypeStruct((NC, N_PAD, D), jnp.float32),
    mesh=_MESH,
    scratch_types=[
        pltpu.VMEM((CHUNK, D), jnp.float32),
        pltpu.VMEM((CHUNK, D), jnp.float32),
        pltpu.VMEM((CHUNK, D), jnp.float32),
        pltpu.VMEM((CHUNK,), jnp.int32),
        pltpu.VMEM_SHARED((N_PAD, D), jnp.float32),
        pltpu.SemaphoreType.DMA,
        pltpu.SemaphoreType.DMA,
        pltpu.SemaphoreType.DMA,
    ],
)(_fc_body)


def _fwd_matmuls_kernel(ph_ref, uf_ref, x_ref, wf_ref, bf_ref, fh_ref, fx_ref):
    i = pl.program_id(0)
    fh_ref[...] = jnp.dot(
        ph_ref[...], uf_ref[...], preferred_element_type=jnp.float32
    )

    @pl.when(i < 10)
    def _():
        fx_ref[...] = (
            jnp.dot(x_ref[...], wf_ref[...], preferred_element_type=jnp.float32)
            + bf_ref[...]
        )


def _final_kernel(x_ref, htp_ref, fcp_ref, wc_ref, bc_ref, c_ref, h_ref):
    ht = htp_ref[0] + htp_ref[1]
    big_in = jnp.concatenate([x_ref[...], ht], axis=1)
    big = (
        jnp.dot(big_in, wc_ref[...], preferred_element_type=jnp.float32)
        + bc_ref[...]
    )
    z_i = big[:, :D]
    z_o = big[:, D : 2 * D]
    z_u = big[:, 2 * D :]
    fc = fcp_ref[0] + fcp_ref[1]
    cc = jax.nn.sigmoid(z_i) * jnp.tanh(z_u) + fc
    c_ref[...] = cc
    h_ref[...] = jax.nn.sigmoid(z_o) * jnp.tanh(cc)


@jax.jit
def kernel(inputs_mat, prev_c_mat, prev_h_mat, segment_ids, W_combined,
           b_combined, W_f, U_f, b_f):
    seg = segment_ids.astype(jnp.int32)

    htp = _seg_sum_h(prev_h_mat, seg)

    f_h, f_x = pl.pallas_call(
        _fwd_matmuls_kernel,
        grid=(100,),
        in_specs=[
            pl.BlockSpec((1600, D), lambda i: (i, 0)),
            pl.BlockSpec((D, D), lambda i: (0, 0)),
            pl.BlockSpec((1000, D), lambda i: (jnp.minimum(i, 9), 0)),
            pl.BlockSpec((D, D), lambda i: (0, 0)),
            pl.BlockSpec((1, D), lambda i: (0, 0)),
        ],
        out_specs=[
            pl.BlockSpec((1600, D), lambda i: (i, 0)),
            pl.BlockSpec((1000, D), lambda i: (jnp.minimum(i, 9), 0)),
        ],
        out_shape=[
            jax.ShapeDtypeStruct((N_EDGES, D), jnp.float32),
            jax.ShapeDtypeStruct((N_NODES, D), jnp.float32),
        ],
    )(prev_h_mat, U_f, inputs_mat, W_f, b_f)

    fcp = _fc_partials(f_h, prev_c_mat, seg, f_x)

    c, h = pl.pallas_call(
        _final_kernel,
        grid=(10,),
        in_specs=[
            pl.BlockSpec((1000, D), lambda i: (i, 0)),
            pl.BlockSpec((NC, 1000, D), lambda i: (0, i, 0)),
            pl.BlockSpec((NC, 1000, D), lambda i: (0, i, 0)),
            pl.BlockSpec((2 * D, 3 * D), lambda i: (0, 0)),
            pl.BlockSpec((1, 3 * D), lambda i: (0, 0)),
        ],
        out_specs=[
            pl.BlockSpec((1000, D), lambda i: (i, 0)),
            pl.BlockSpec((1000, D), lambda i: (i, 0)),
        ],
        out_shape=[
            jax.ShapeDtypeStruct((N_NODES, D), jnp.float32),
            jax.ShapeDtypeStruct((N_NODES, D), jnp.float32),
        ],
    )(inputs_mat, htp, fcp, W_combined, b_combined)

    return (c, h)


# speedup vs baseline: 2.4240x; 1.0027x over previous
import functools

import jax
import jax.numpy as jnp
from jax import lax
from jax.experimental import pallas as pl
from jax.experimental.pallas import tpu as pltpu
from jax.experimental.pallas import tpu_sc as plsc

N_NODES = 10000
N_EDGES = 160000
D = 128
LANES = 16

NC = 2
NS = 16
NW = NC * NS
CHUNK = 128
N_CHUNKS = N_EDGES // CHUNK
MAX_T = (N_CHUNKS + NW - 1) // NW
N_PAD = 10112
ROWS_PER_SUB = N_PAD // NS

_MESH = plsc.VectorSubcoreMesh(core_axis_name="c", subcore_axis_name="s")


def _zero_accumulator(rows_v, acc_sh, s, stage_rows):
    def zrow(i, carry):
        for j in range(D // LANES):
            rows_v[i, pl.ds(j * LANES, LANES)] = jnp.zeros((LANES,), jnp.float32)
        return carry

    lax.fori_loop(0, stage_rows, zrow, 0)
    nfull, rem = divmod(ROWS_PER_SUB, stage_rows)
    for z in range(nfull):
        pltpu.sync_copy(
            rows_v,
            acc_sh.at[pl.ds(s * ROWS_PER_SUB + z * stage_rows, stage_rows)],
        )
    if rem:
        pltpu.sync_copy(
            rows_v.at[pl.ds(0, rem)],
            acc_sh.at[pl.ds(s * ROWS_PER_SUB + nfull * stage_rows, rem)],
        )


def _seg_sum_h_body(prev_h, seg, out, rows_v, idx_v, acc_sh, sem_ld, sem_ix, sem_sc):
    c = lax.axis_index("c")
    s = lax.axis_index("s")
    w = c * NS + s

    _zero_accumulator(rows_v.at[0], acc_sh, s, CHUNK)
    plsc.subcore_barrier()

    def start_loads(t, b):
        k = w + NW * t

        @pl.when(k < N_CHUNKS)
        def _():
            base = k * CHUNK
            pltpu.async_copy(prev_h.at[pl.ds(base, CHUNK)], rows_v.at[b], sem_ld.at[b])
            pltpu.async_copy(seg.at[pl.ds(base, CHUNK)], idx_v.at[b], sem_ix.at[b])

    start_loads(0, 0)

    def pair(t2, carry):
        for b in (0, 1):
            t = 2 * t2 + b
            k = w + NW * t

            @pl.when(k < N_CHUNKS)
            def _():
                pltpu.make_async_copy(
                    prev_h.at[pl.ds(0, CHUNK)], rows_v.at[b], sem_ld.at[b]
                ).wait()
                pltpu.make_async_copy(
                    seg.at[pl.ds(0, CHUNK)], idx_v.at[b], sem_ix.at[b]
                ).wait()

            @pl.when((t >= 1) & (k - NW < N_CHUNKS))
            def _():
                pltpu.make_async_copy(
                    rows_v.at[1 - b], acc_sh.at[idx_v.at[1 - b]], sem_sc.at[1 - b]
                ).wait()

            start_loads(t + 1, 1 - b)

            @pl.when(k < N_CHUNKS)
            def _():
                pltpu.async_copy(
                    rows_v.at[b], acc_sh.at[idx_v.at[b]], sem_sc.at[b], add=True
                )

        return carry

    lax.fori_loop(0, MAX_T // 2, pair, 0)
    @pl.when(w + NW * (MAX_T - 1) < N_CHUNKS)
    def _():
        pltpu.make_async_copy(
            rows_v.at[1], acc_sh.at[idx_v.at[1]], sem_sc.at[1]
        ).wait()

    plsc.subcore_barrier()
    pltpu.sync_copy(
        acc_sh.at[pl.ds(s * ROWS_PER_SUB, ROWS_PER_SUB)],
        out.at[c, pl.ds(s * ROWS_PER_SUB, ROWS_PER_SUB)],
    )


_seg_sum_h = functools.partial(
    pl.kernel,
    out_type=jax.ShapeDtypeStruct((NC, N_PAD, D), jnp.float32),
    mesh=_MESH,
    scratch_types=[
        pltpu.VMEM((2, CHUNK, D), jnp.float32),
        pltpu.VMEM((2, CHUNK), jnp.int32),
        pltpu.VMEM_SHARED((N_PAD, D), jnp.float32),
        pltpu.SemaphoreType.DMA((2,)),
        pltpu.SemaphoreType.DMA((2,)),
        pltpu.SemaphoreType.DMA((2,)),
    ],
)(_seg_sum_h_body)


def _fc_body(fh, pc, seg, fx, out, fh_v, pc_v, fxg_v, idx_v, acc_sh,
             sem, sem2, sem3, sem4):
    c = lax.axis_index("c")
    s = lax.axis_index("s")
    w = c * NS + s

    H = CHUNK // 2

    _zero_accumulator(fh_v, acc_sh, s, CHUNK)
    plsc.subcore_barrier()

    def body(t, carry):
        k = w + NW * t

        @pl.when(k < N_CHUNKS)
        def _():
            base = k * CHUNK
            cp1 = pltpu.async_copy(fh.at[pl.ds(base, CHUNK)], fh_v, sem)
            cp2 = pltpu.async_copy(pc.at[pl.ds(base, CHUNK)], pc_v, sem2)
            pltpu.sync_copy(seg.at[pl.ds(base, CHUNK)], idx_v)
            cp3 = pltpu.async_copy(
                fx.at[idx_v.at[pl.ds(0, H)]], fxg_v.at[pl.ds(0, H)], sem3
            )
            cp4 = pltpu.async_copy(
                fx.at[idx_v.at[pl.ds(H, H)]], fxg_v.at[pl.ds(H, H)], sem4
            )
            cp1.wait()
            cp2.wait()
            cp3.wait()

            @plsc.parallel_loop(0, H, 1, unroll=4)
            def row(i):
                for j in range(D // LANES):
                    sl = pl.ds(j * LANES, LANES)
                    x = fh_v[i, sl] + fxg_v[i, sl]
                    sg = 1.0 / (1.0 + jnp.exp(-x))
                    fh_v[i, sl] = sg * pc_v[i, sl]

            cp4.wait()

            @plsc.parallel_loop(H, CHUNK, 1, unroll=4)
            def row2(i):
                for j in range(D // LANES):
                    sl = pl.ds(j * LANES, LANES)
                    x = fh_v[i, sl] + fxg_v[i, sl]
                    sg = 1.0 / (1.0 + jnp.exp(-x))
                    fh_v[i, sl] = sg * pc_v[i, sl]

            pltpu.sync_copy(fh_v, acc_sh.at[idx_v], add=True)

        return carry

    lax.fori_loop(0, MAX_T, body, 0)
    plsc.subcore_barrier()
    pltpu.sync_copy(
        acc_sh.at[pl.ds(s * ROWS_PER_SUB, ROWS_PER_SUB)],
        out.at[c, pl.ds(s * ROWS_PER_SUB, ROWS_PER_SUB)],
    )


_fc_partials = functools.partial(
    pl.kernel,
    out_type=jax.ShapeDtypeStruct((NC, N_PAD, D), jnp.float32),
    mesh=_MESH,
    scratch_types=[
        pltpu.VMEM((CHUNK, D), jnp.float32),
        pltpu.VMEM((CHUNK, D), jnp.float32),
        pltpu.VMEM((CHUNK, D), jnp.float32),
        pltpu.VMEM((CHUNK,), jnp.int32),
        pltpu.VMEM_SHARED((N_PAD, D), jnp.float32),
        pltpu.SemaphoreType.DMA,
        pltpu.SemaphoreType.DMA,
        pltpu.SemaphoreType.DMA,
        pltpu.SemaphoreType.DMA,
    ],
)(_fc_body)


def _fwd_matmuls_kernel(ph_ref, uf_ref, x_ref, wf_ref, bf_ref, fh_ref, fx_ref):
    i = pl.program_id(0)
    fh_ref[...] = jnp.dot(
        ph_ref[...], uf_ref[...], preferred_element_type=jnp.float32
    )

    @pl.when(i < 10)
    def _():
        fx_ref[...] = (
            jnp.dot(x_ref[...], wf_ref[...], preferred_element_type=jnp.float32)
            + bf_ref[...]
        )


def _final_kernel(x_ref, htp_ref, fcp_ref, wc_ref, bc_ref, c_ref, h_ref):
    ht = htp_ref[0] + htp_ref[1]
    big_in = jnp.concatenate([x_ref[...], ht], axis=1)
    big = (
        jnp.dot(big_in, wc_ref[...], preferred_element_type=jnp.float32)
        + bc_ref[...]
    )
    z_i = big[:, :D]
    z_o = big[:, D : 2 * D]
    z_u = big[:, 2 * D :]
    fc = fcp_ref[0] + fcp_ref[1]
    cc = jax.nn.sigmoid(z_i) * jnp.tanh(z_u) + fc
    c_ref[...] = cc
    h_ref[...] = jax.nn.sigmoid(z_o) * jnp.tanh(cc)


@jax.jit
def kernel(inputs_mat, prev_c_mat, prev_h_mat, segment_ids, W_combined,
           b_combined, W_f, U_f, b_f):
    seg = segment_ids.astype(jnp.int32)

    htp = _seg_sum_h(prev_h_mat, seg)

    f_h, f_x = pl.pallas_call(
        _fwd_matmuls_kernel,
        grid=(100,),
        in_specs=[
            pl.BlockSpec((1600, D), lambda i: (i, 0)),
            pl.BlockSpec((D, D), lambda i: (0, 0)),
            pl.BlockSpec((1000, D), lambda i: (jnp.minimum(i, 9), 0)),
            pl.BlockSpec((D, D), lambda i: (0, 0)),
            pl.BlockSpec((1, D), lambda i: (0, 0)),
        ],
        out_specs=[
            pl.BlockSpec((1600, D), lambda i: (i, 0)),
            pl.BlockSpec((1000, D), lambda i: (jnp.minimum(i, 9), 0)),
        ],
        out_shape=[
            jax.ShapeDtypeStruct((N_EDGES, D), jnp.float32),
            jax.ShapeDtypeStruct((N_NODES, D), jnp.float32),
        ],
    )(prev_h_mat, U_f, inputs_mat, W_f, b_f)

    fcp = _fc_partials(f_h, prev_c_mat, seg, f_x)

    c, h = pl.pallas_call(
        _final_kernel,
        grid=(10,),
        in_specs=[
            pl.BlockSpec((1000, D), lambda i: (i, 0)),
            pl.BlockSpec((NC, 1000, D), lambda i: (0, i, 0)),
            pl.BlockSpec((NC, 1000, D), lambda i: (0, i, 0)),
            pl.BlockSpec((2 * D, 3 * D), lambda i: (0, 0)),
            pl.BlockSpec((1, 3 * D), lambda i: (0, 0)),
        ],
        out_specs=[
            pl.BlockSpec((1000, D), lambda i: (i, 0)),
            pl.BlockSpec((1000, D), lambda i: (i, 0)),
        ],
        out_shape=[
            jax.ShapeDtypeStruct((N_NODES, D), jnp.float32),
            jax.ShapeDtypeStruct((N_NODES, D), jnp.float32),
        ],
    )(inputs_mat, htp, fcp, W_combined, b_combined)

    return (c, h)
